# baseline (speedup 1.0000x reference)
import functools

import jax
import jax.numpy as jnp
from jax.experimental import pallas as pl
from jax.experimental.pallas import tpu as pltpu

N = 1024
E = 2048


def _wd_body(h_ref, w2_ref, b2_ref, out_ref):
    acc = jax.lax.dot_general(
        h_ref[...], w2_ref[...], (((1,), (0,)), ((), ())),
        preferred_element_type=jnp.float32)
    out_ref[...] = (acc + b2_ref[...]).astype(jnp.bfloat16)


def _wd_bf16(h_bf16, W2_bf16, b2, cb):
    k, m = W2_bf16.shape
    b2r = b2.reshape(1, m)
    return pl.pallas_call(
        _wd_body,
        grid=(m // cb,),
        in_specs=[
            pl.BlockSpec((E, k), lambda j: (0, 0)),
            pl.BlockSpec((k, cb), lambda j: (0, j)),
            pl.BlockSpec((1, cb), lambda j: (0, j)),
        ],
        out_specs=pl.BlockSpec((E, cb), lambda j: (0, j)),
        out_shape=jax.ShapeDtypeStruct((E, m), jnp.bfloat16),
        compiler_params=pltpu.CompilerParams(
            dimension_semantics=("arbitrary",)),
    )(h_bf16, W2_bf16, b2r)


def _nnconv(x, src, dst, edge_attr, W1, b1, W2, b2, root, bias, in_ch, out_ch):
    h = jax.nn.leaky_relu(jnp.dot(edge_attr, W1) + b1, negative_slope=0.01)
    Wd = _wd_bf16(h.astype(jnp.bfloat16), W2.astype(jnp.bfloat16), b2,
                  cb=2048).reshape(E, in_ch, out_ch)
    x_j = jnp.take(x, src, axis=0)
    msg = jnp.einsum('ei,eio->eo', x_j.astype(jnp.bfloat16), Wd,
                     preferred_element_type=jnp.float32)
    s = jax.ops.segment_sum(msg, dst, num_segments=x.shape[0])
    c = jax.ops.segment_sum(jnp.ones((msg.shape[0],), dtype=msg.dtype), dst,
                            num_segments=x.shape[0])
    mean = s / jnp.maximum(c, 1.0)[:, None]
    return mean + jnp.dot(x, root) + bias


def kernel(x, edge_index, edge_attr, epoch,
           nn1_W1, nn1_b1, nn1_W2, nn1_b2, root1, bias1,
           nn2_W1, nn2_b1, nn2_W2, nn2_b2, root2, bias2,
           nn3_W1, nn3_b1, nn3_W2, nn3_b2, root3, bias3):
    tau = 500.0 / (epoch + 1)
    src = edge_index[0]
    dst = edge_index[1]
    d1 = jax.nn.leaky_relu(_nnconv(x, src, dst, edge_attr, nn1_W1, nn1_b1, nn1_W2, nn1_b2, root1, bias1, 64, 512), 0.01)
    d2 = jax.nn.leaky_relu(_nnconv(d1, src, dst, edge_attr, nn2_W1, nn2_b1, nn2_W2, nn2_b2, root2, bias2, 512, 256), 0.01)
    d3 = jax.nn.leaky_relu(_nnconv(d2, src, dst, edge_attr, nn3_W1, nn3_b1, nn3_W2, nn3_b2, root3, bias3, 256, 64), 0.01)
    g = jax.random.gumbel(jax.random.key(42), d3.shape, dtype=d3.dtype)
    y_soft = jax.nn.softmax((d3 + g) / tau, axis=-1)
    y_hard = jax.nn.one_hot(jnp.argmax(y_soft, axis=-1), d3.shape[-1], dtype=d3.dtype)
    return y_hard - jax.lax.stop_gradient(y_soft) + y_soft


# speedup vs baseline: 1.0996x; 1.0996x over previous
import functools

import jax
import jax.numpy as jnp
from jax.experimental import pallas as pl
from jax.experimental.pallas import tpu as pltpu

N = 1024
E = 2048


def _wd_body3(h_ref, w2_ref, b2_ref, out_ref, *, ib, out_ch):
    k = h_ref.shape[1]
    eb = h_ref.shape[0]
    w2 = w2_ref[...].reshape(k, ib * out_ch)
    acc = jax.lax.dot_general(
        h_ref[...], w2, (((1,), (0,)), ((), ())),
        preferred_element_type=jnp.float32)
    acc = acc.reshape(eb, ib, out_ch) + b2_ref[...][None, :, :]
    out_ref[...] = acc.astype(jnp.bfloat16)


def _wd_bf16(h_bf16, W2_bf16, b2, in_ch, out_ch, ib, eb):
    k = W2_bf16.shape[0]
    w2r = W2_bf16.reshape(k, in_ch, out_ch)
    b2r = b2.reshape(in_ch, out_ch)
    return pl.pallas_call(
        functools.partial(_wd_body3, ib=ib, out_ch=out_ch),
        grid=(in_ch // ib, E // eb),
        in_specs=[
            pl.BlockSpec((eb, k), lambda i, e: (e, 0)),
            pl.BlockSpec((k, ib, out_ch), lambda i, e: (0, i, 0)),
            pl.BlockSpec((ib, out_ch), lambda i, e: (i, 0)),
        ],
        out_specs=pl.BlockSpec((eb, ib, out_ch), lambda i, e: (e, i, 0)),
        out_shape=jax.ShapeDtypeStruct((E, in_ch, out_ch), jnp.bfloat16),
        compiler_params=pltpu.CompilerParams(
            dimension_semantics=("arbitrary", "arbitrary")),
    )(h_bf16, w2r, b2r)


def _nnconv(x, src, dst, edge_attr, W1, b1, W2, b2, root, bias, in_ch, out_ch,
            ib, eb):
    h = jax.nn.leaky_relu(jnp.dot(edge_attr, W1) + b1, negative_slope=0.01)
    Wd = _wd_bf16(h.astype(jnp.bfloat16), W2.astype(jnp.bfloat16), b2,
                  in_ch, out_ch, ib, eb)
    x_j = jnp.take(x, src, axis=0)
    msg = jnp.einsum('ei,eio->eo', x_j.astype(jnp.bfloat16), Wd,
                     preferred_element_type=jnp.float32)
    s = jax.ops.segment_sum(msg, dst, num_segments=x.shape[0])
    c = jax.ops.segment_sum(jnp.ones((msg.shape[0],), dtype=msg.dtype), dst,
                            num_segments=x.shape[0])
    mean = s / jnp.maximum(c, 1.0)[:, None]
    return mean + jnp.dot(x, root) + bias


def kernel(x, edge_index, edge_attr, epoch,
           nn1_W1, nn1_b1, nn1_W2, nn1_b2, root1, bias1,
           nn2_W1, nn2_b1, nn2_W2, nn2_b2, root2, bias2,
           nn3_W1, nn3_b1, nn3_W2, nn3_b2, root3, bias3):
    tau = 500.0 / (epoch + 1)
    src = edge_index[0]
    dst = edge_index[1]
    d1 = jax.nn.leaky_relu(_nnconv(x, src, dst, edge_attr, nn1_W1, nn1_b1, nn1_W2, nn1_b2, root1, bias1, 64, 512, ib=16, eb=512), 0.01)
    d2 = jax.nn.leaky_relu(_nnconv(d1, src, dst, edge_attr, nn2_W1, nn2_b1, nn2_W2, nn2_b2, root2, bias2, 512, 256, ib=16, eb=512), 0.01)
    d3 = jax.nn.leaky_relu(_nnconv(d2, src, dst, edge_attr, nn3_W1, nn3_b1, nn3_W2, nn3_b2, root3, bias3, 256, 64, ib=16, eb=512), 0.01)
    g = jax.random.gumbel(jax.random.key(42), d3.shape, dtype=d3.dtype)
    y_soft = jax.nn.softmax((d3 + g) / tau, axis=-1)
    y_hard = jax.nn.one_hot(jnp.argmax(y_soft, axis=-1), d3.shape[-1], dtype=d3.dtype)
    return y_hard - jax.lax.stop_gradient(y_soft) + y_soft
